# baseline (speedup 1.0000x reference)
import functools

import jax
import jax.numpy as jnp
from jax import lax
from jax.experimental import pallas as pl
from jax.experimental.pallas import tpu as pltpu
from jax.experimental.pallas import tpu_sc as plsc

N = 10000
DEG = 32
DYN = 8
LAT = 128
HID = 16
E = N * DEG

NC = 2
NS = 16
NW = NC * NS

CHUNK = 125
EPT = E // NW
NCHUNK = EPT // CHUNK
NBUF = 5
NG = NCHUNK // NBUF
NPAD = -(-N // (NS * 8)) * NS * 8
ZROWS = NPAD // NS
OTILES = 10
OROWS = N // OTILES

_mesh = plsc.VectorSubcoreMesh(core_axis_name="c", subcore_axis_name="s")


@functools.partial(
    pl.kernel,
    out_type=jax.ShapeDtypeStruct((NC, N, HID), jnp.float32),
    mesh=_mesh,
    scratch_types=[
        pltpu.VMEM((NCHUNK, CHUNK), jnp.int32),
        pltpu.VMEM((NCHUNK, CHUNK), jnp.int32),
        pltpu.VMEM((NBUF, CHUNK, HID), jnp.float32),
        pltpu.VMEM((ZROWS, HID), jnp.float32),
        pltpu.VMEM_SHARED((NPAD, HID), jnp.float32),
        [pltpu.SemaphoreType.DMA] * NBUF,
        [pltpu.SemaphoreType.DMA] * NBUF,
    ],
    compiler_params=pltpu.CompilerParams(use_tc_tiling_on_sc=False),
)
def _sc_route(p_hbm, gidx_hbm, didx_hbm, out_hbm,
              gidx_v, didx_v, rows_v, zbuf_v, acc_sh, gsems, ssems):
    c = lax.axis_index("c")
    s = lax.axis_index("s")
    w = s * NC + c
    pltpu.sync_copy(gidx_hbm.at[w], gidx_v)
    pltpu.sync_copy(didx_hbm.at[w], didx_v)
    def zrow(r, carry):
        zbuf_v[r, :] = jnp.zeros((HID,), jnp.float32)
        return carry
    lax.fori_loop(0, ZROWS, zrow, 0)
    pltpu.sync_copy(zbuf_v, acc_sh.at[pl.ds(s * ZROWS, ZROWS)])
    plsc.subcore_barrier()

    def gather_start(j, b):
        pltpu.async_copy(p_hbm.at[gidx_v.at[j]], rows_v.at[b], gsems[b])

    def gather_wait(j, b):
        pltpu.make_async_copy(p_hbm.at[gidx_v.at[j]], rows_v.at[b],
                              gsems[b]).wait()

    def scat_start(j, b):
        pltpu.async_copy(rows_v.at[b], acc_sh.at[didx_v.at[j]], ssems[b],
                         add=True)

    def scat_wait(j, b):
        pltpu.make_async_copy(rows_v.at[b], acc_sh.at[didx_v.at[j]],
                              ssems[b]).wait()

    for b in range(NBUF):
        gather_start(b, b)

    def group(i, carry):
        j0 = i * NBUF
        for b in range(NBUF):
            gather_wait(j0 + b, b)
            scat_start(j0 + b, b)
        for b in range(NBUF):
            scat_wait(j0 + b, b)
            gather_start(j0 + NBUF + b, b)
        return carry

    lax.fori_loop(0, NG - 1, group, 0)

    jl = (NG - 1) * NBUF
    for b in range(NBUF):
        gather_wait(jl + b, b)
        scat_start(jl + b, b)
    for b in range(NBUF):
        scat_wait(jl + b, b)

    plsc.subcore_barrier()
    @pl.when(s < OTILES)
    def _():
        pltpu.sync_copy(acc_sh.at[pl.ds(s * OROWS, OROWS)],
                        out_hbm.at[c, pl.ds(s * OROWS, OROWS)])


def _mm_body(x_ref, w_ref, dyn_ref, wpd_ref, bpre_ref, o_ref, pd_ref):
    o_ref[...] = jnp.dot(x_ref[...], w_ref[...],
                         preferred_element_type=jnp.float32)
    pd_ref[...] = (jnp.dot(dyn_ref[...], wpd_ref[...],
                           preferred_element_type=jnp.float32)
                   + bpre_ref[...])


_BN1 = 1000


def _project(pk_lat_out, w_cat, dyn_in, w_pre_dyn, b_pre):
    return pl.pallas_call(
        _mm_body,
        grid=(N // _BN1,),
        in_specs=[
            pl.BlockSpec((_BN1, LAT), lambda i: (i, 0)),
            pl.BlockSpec((LAT, DEG * HID), lambda i: (0, 0)),
            pl.BlockSpec((_BN1, DYN), lambda i: (i, 0)),
            pl.BlockSpec((DYN, HID), lambda i: (0, 0)),
            pl.BlockSpec((1, HID), lambda i: (0, 0)),
        ],
        out_specs=[
            pl.BlockSpec((_BN1, DEG * HID), lambda i: (i, 0)),
            pl.BlockSpec((_BN1, HID), lambda i: (i, 0)),
        ],
        out_shape=[
            jax.ShapeDtypeStruct((N, DEG * HID), jnp.float32),
            jax.ShapeDtypeStruct((N, HID), jnp.float32),
        ],
    )(pk_lat_out, w_cat, dyn_in, w_pre_dyn, b_pre.reshape(1, HID))


def _tail_body(acc_ref, pd_ref, c_ref, h_ref, wg_ref, blstm_ref,
               wout_ref, bdyn_ref, blat_ref,
               dyn_out_ref, lat_out_ref, cnew_ref, hnew_ref):
    pre = jnp.tanh(acc_ref[0] + acc_ref[1] + pd_ref[...])
    u = jnp.concatenate([pre, h_ref[...]], axis=1)
    gates = (jnp.dot(u, wg_ref[...], preferred_element_type=jnp.float32)
             + blstm_ref[...])
    i_g = jax.nn.sigmoid(gates[:, 0:HID])
    f_g = jax.nn.sigmoid(gates[:, HID:2 * HID])
    g_g = jnp.tanh(gates[:, 2 * HID:3 * HID])
    o_g = jax.nn.sigmoid(gates[:, 3 * HID:4 * HID])
    c_new = f_g * c_ref[...] + i_g * g_g
    h_new = o_g * jnp.tanh(c_new)
    cnew_ref[...] = c_new
    hnew_ref[...] = h_new
    vw = jnp.dot(h_new, wout_ref[...], preferred_element_type=jnp.float32)
    dyn_out_ref[...] = vw[:, 0:DYN] + bdyn_ref[...]
    lat_out_ref[...] = jnp.tanh(vw[:, LAT:2 * LAT] + blat_ref[...])


_BN2 = 2000


def _tail(acc, predyn, pk_lstm_c, pk_lstm_h, w_gates, b_lstm, w_out,
          b_dyn, b_lat):
    g = N // _BN2
    full = lambda r, c: pl.BlockSpec((r, c), lambda i: (0, 0))
    row = lambda c: pl.BlockSpec((_BN2, c), lambda i: (i, 0))
    return pl.pallas_call(
        _tail_body,
        grid=(g,),
        in_specs=[
            pl.BlockSpec((NC, _BN2, HID), lambda i: (0, i, 0)),
            row(HID), row(HID), row(HID),
            full(2 * HID, 4 * HID), full(1, 4 * HID),
            full(HID, 2 * LAT), full(1, DYN), full(1, LAT),
        ],
        out_specs=[row(DYN), row(LAT), row(HID), row(HID)],
        out_shape=[
            jax.ShapeDtypeStruct((N, DYN), jnp.float32),
            jax.ShapeDtypeStruct((N, LAT), jnp.float32),
            jax.ShapeDtypeStruct((N, HID), jnp.float32),
            jax.ShapeDtypeStruct((N, HID), jnp.float32),
        ],
    )(acc, predyn, pk_lstm_c, pk_lstm_h, w_gates,
      b_lstm.reshape(1, 4 * HID), w_out, b_dyn.reshape(1, DYN),
      b_lat.reshape(1, LAT))


def kernel(dyn_in, pk_lat_out, pk_lstm_c, pk_lstm_h, edge_src, edge_dst,
           edge_slot, W_pre, b_pre, W_ih, W_hh, b_lstm, W_dyn, b_dyn,
           W_lat, b_lat):
    w_cat = (W_pre[DYN:].reshape(DEG, LAT, HID)
             .transpose(1, 0, 2).reshape(LAT, DEG * HID))
    w_pre_dyn = W_pre[:DYN]
    w_gates = jnp.concatenate([W_ih, W_hh], axis=0)
    w_out = jnp.zeros((HID, 2 * LAT), jnp.float32)
    w_out = w_out.at[:, 0:DYN].set(W_dyn).at[:, LAT:2 * LAT].set(W_lat)

    gidx = (edge_src * DEG + edge_slot).reshape(NW, NCHUNK, CHUNK)
    didx = edge_dst.reshape(NW, NCHUNK, CHUNK)

    p, predyn = _project(pk_lat_out, w_cat, dyn_in, w_pre_dyn, b_pre)
    acc = _sc_route(p.reshape(N * DEG, HID), gidx, didx)
    return _tail(acc, predyn, pk_lstm_c, pk_lstm_h, w_gates, b_lstm,
                 w_out, b_dyn, b_lat)


# speedup vs baseline: 1.1143x; 1.1143x over previous
import functools

import jax
import jax.numpy as jnp
from jax import lax
from jax.experimental import pallas as pl
from jax.experimental.pallas import tpu as pltpu
from jax.experimental.pallas import tpu_sc as plsc

N = 10000
DEG = 32
DYN = 8
LAT = 128
HID = 16
E = N * DEG

NC = 2
NS = 16
NW = NC * NS

CHUNK = 125
EPT = E // NW
NCHUNK = EPT // CHUNK
NBUF = 5
NG = NCHUNK // NBUF
NPAD = -(-N // (NS * 8)) * NS * 8
ZROWS = NPAD // NS
OTILES = 10
OROWS = N // OTILES

_mesh = plsc.VectorSubcoreMesh(core_axis_name="c", subcore_axis_name="s")


@functools.partial(
    pl.kernel,
    out_type=jax.ShapeDtypeStruct((NC, N, HID), jnp.float32),
    mesh=_mesh,
    scratch_types=[
        pltpu.VMEM((NCHUNK, CHUNK), jnp.int32),
        pltpu.VMEM((NCHUNK, CHUNK), jnp.int32),
        pltpu.VMEM((NBUF, CHUNK, HID), jnp.float32),
        pltpu.VMEM((ZROWS, HID), jnp.float32),
        pltpu.VMEM_SHARED((NPAD, HID), jnp.float32),
        [pltpu.SemaphoreType.DMA] * NBUF,
        [pltpu.SemaphoreType.DMA] * NBUF,
    ],
    compiler_params=pltpu.CompilerParams(use_tc_tiling_on_sc=False),
)
def _sc_route(p_hbm, gidx_hbm, didx_hbm, out_hbm,
              gidx_v, didx_v, rows_v, zbuf_v, acc_sh, gsems, ssems):
    c = lax.axis_index("c")
    s = lax.axis_index("s")
    w = s * NC + c
    pltpu.sync_copy(gidx_hbm.at[w], gidx_v)
    pltpu.sync_copy(didx_hbm.at[w], didx_v)
    def zrow(r, carry):
        zbuf_v[r, :] = jnp.zeros((HID,), jnp.float32)
        return carry
    lax.fori_loop(0, ZROWS, zrow, 0)
    pltpu.sync_copy(zbuf_v, acc_sh.at[pl.ds(s * ZROWS, ZROWS)])
    plsc.subcore_barrier()

    def gather_start(j, b):
        pltpu.async_copy(p_hbm.at[gidx_v.at[j]], rows_v.at[b], gsems[b])

    def gather_wait(j, b):
        pltpu.make_async_copy(p_hbm.at[gidx_v.at[j]], rows_v.at[b],
                              gsems[b]).wait()

    def scat_start(j, b):
        pltpu.async_copy(rows_v.at[b], acc_sh.at[didx_v.at[j]], ssems[b],
                         add=True)

    def scat_wait(j, b):
        pltpu.make_async_copy(rows_v.at[b], acc_sh.at[didx_v.at[j]],
                              ssems[b]).wait()

    for b in range(NBUF):
        gather_start(b, b)

    def group(i, carry):
        j0 = i * NBUF
        for b in range(NBUF):
            gather_wait(j0 + b, b)
            scat_start(j0 + b, b)
        for b in range(NBUF):
            scat_wait(j0 + b, b)
            gather_start(j0 + NBUF + b, b)
        return carry

    lax.fori_loop(0, NG - 1, group, 0)

    jl = (NG - 1) * NBUF
    for b in range(NBUF):
        gather_wait(jl + b, b)
        scat_start(jl + b, b)
    for b in range(NBUF):
        scat_wait(jl + b, b)

    plsc.subcore_barrier()
    @pl.when(s < OTILES)
    def _():
        pltpu.sync_copy(acc_sh.at[pl.ds(s * OROWS, OROWS)],
                        out_hbm.at[c, pl.ds(s * OROWS, OROWS)])


def _mm_body(x_ref, w_ref, o_ref):
    o_ref[...] = jnp.dot(x_ref[...], w_ref[...],
                         preferred_element_type=jnp.float32)


_BN1 = 1000


def _project(pk_lat_out, w_cat):
    return pl.pallas_call(
        _mm_body,
        grid=(N // _BN1,),
        in_specs=[
            pl.BlockSpec((_BN1, LAT), lambda i: (i, 0)),
            pl.BlockSpec((LAT, DEG * HID), lambda i: (0, 0)),
        ],
        out_specs=pl.BlockSpec((_BN1, DEG * HID), lambda i: (i, 0)),
        out_shape=jax.ShapeDtypeStruct((N, DEG * HID), jnp.float32),
    )(pk_lat_out, w_cat)


PK = 8
NR = N // PK


def _tail_body(acc_ref, dyn_ref, c_ref, h_ref, wd_ref, bpre_ref, wih_ref,
               whh_ref, blstm_ref, wdyn_ref, bdyn_ref, wlat_ref, blat_ref,
               dyn_out_ref, lat_out_ref, cnew_ref, hnew_ref):
    acc = acc_ref[0] + acc_ref[1]
    predyn = jnp.dot(dyn_ref[...], wd_ref[...],
                     preferred_element_type=jnp.float32)
    pre = jnp.tanh(acc + predyn + bpre_ref[...])
    gates = (jnp.dot(pre, wih_ref[...], preferred_element_type=jnp.float32)
             + jnp.dot(h_ref[...], whh_ref[...],
                       preferred_element_type=jnp.float32)
             + blstm_ref[...])
    i_g = jax.nn.sigmoid(gates[:, 0:LAT])
    f_g = jax.nn.sigmoid(gates[:, LAT:2 * LAT])
    g_g = jnp.tanh(gates[:, 2 * LAT:3 * LAT])
    o_g = jax.nn.sigmoid(gates[:, 3 * LAT:4 * LAT])
    c_new = f_g * c_ref[...] + i_g * g_g
    h_new = o_g * jnp.tanh(c_new)
    cnew_ref[...] = c_new
    hnew_ref[...] = h_new
    dyn_out_ref[...] = (jnp.dot(h_new, wdyn_ref[...],
                                preferred_element_type=jnp.float32)
                        + bdyn_ref[...])
    lat_out_ref[...] = jnp.tanh(jnp.dot(h_new, wlat_ref[...],
                                        preferred_element_type=jnp.float32)
                                + blat_ref[...])


def _tail(acc, dyn_in, pk_lstm_c, pk_lstm_h, w_pre_dyn, b_pre, w_ih, w_hh,
          b_lstm, w_dyn, b_dyn, w_lat, b_lat):
    eye = jnp.eye(PK, dtype=jnp.float32)
    wd_bd = jnp.einsum('rc,kj->krjc', w_pre_dyn, eye).reshape(PK * DYN,
                                                              PK * HID)
    w4 = w_ih.reshape(HID, 4, HID)
    wih_bd = jnp.einsum('rtc,kj->krtjc', w4, eye).reshape(PK * HID, 4 * LAT)
    w4h = w_hh.reshape(HID, 4, HID)
    whh_bd = jnp.einsum('rtc,kj->krtjc', w4h, eye).reshape(PK * HID, 4 * LAT)
    wdyn_bd = jnp.einsum('rc,kj->krjc', w_dyn, eye).reshape(PK * HID,
                                                            PK * DYN)
    wlat_bd = jnp.einsum('rc,kj->krjc', w_lat, eye).reshape(PK * HID,
                                                            PK * LAT)
    bpre_p = jnp.tile(b_pre, PK).reshape(1, PK * HID)
    blstm_p = jnp.tile(b_lstm.reshape(4, 1, HID),
                       (1, PK, 1)).reshape(1, 4 * LAT)
    bdyn_p = jnp.tile(b_dyn, PK).reshape(1, PK * DYN)
    blat_p = jnp.tile(b_lat, PK).reshape(1, PK * LAT)

    full = lambda r, c: pl.BlockSpec((r, c), lambda: (0, 0))
    outs = pl.pallas_call(
        _tail_body,
        in_specs=[
            pl.BlockSpec((NC, NR, PK * HID), lambda: (0, 0, 0)),
            full(NR, PK * DYN), full(NR, PK * HID), full(NR, PK * HID),
            full(PK * DYN, PK * HID), full(1, PK * HID),
            full(PK * HID, 4 * LAT), full(PK * HID, 4 * LAT),
            full(1, 4 * LAT), full(PK * HID, PK * DYN), full(1, PK * DYN),
            full(PK * HID, PK * LAT), full(1, PK * LAT),
        ],
        out_specs=[full(NR, PK * DYN), full(NR, PK * LAT),
                   full(NR, PK * HID), full(NR, PK * HID)],
        out_shape=[
            jax.ShapeDtypeStruct((NR, PK * DYN), jnp.float32),
            jax.ShapeDtypeStruct((NR, PK * LAT), jnp.float32),
            jax.ShapeDtypeStruct((NR, PK * HID), jnp.float32),
            jax.ShapeDtypeStruct((NR, PK * HID), jnp.float32),
        ],
    )(acc.reshape(NC, NR, PK * HID), dyn_in.reshape(NR, PK * DYN),
      pk_lstm_c.reshape(NR, PK * HID), pk_lstm_h.reshape(NR, PK * HID),
      wd_bd, bpre_p, wih_bd, whh_bd, blstm_p, wdyn_bd, bdyn_p,
      wlat_bd, blat_p)
    dyn_out, lat_out, c_new, h_new = outs
    return (dyn_out.reshape(N, DYN), lat_out.reshape(N, LAT),
            c_new.reshape(N, HID), h_new.reshape(N, HID))


def kernel(dyn_in, pk_lat_out, pk_lstm_c, pk_lstm_h, edge_src, edge_dst,
           edge_slot, W_pre, b_pre, W_ih, W_hh, b_lstm, W_dyn, b_dyn,
           W_lat, b_lat):
    w_cat = (W_pre[DYN:].reshape(DEG, LAT, HID)
             .transpose(1, 0, 2).reshape(LAT, DEG * HID))
    w_pre_dyn = W_pre[:DYN]

    gidx = (edge_src * DEG + edge_slot).reshape(NW, NCHUNK, CHUNK)
    didx = edge_dst.reshape(NW, NCHUNK, CHUNK)

    p = _project(pk_lat_out, w_cat)
    acc = _sc_route(p.reshape(N * DEG, HID), gidx, didx)
    return _tail(acc, dyn_in, pk_lstm_c, pk_lstm_h, w_pre_dyn, b_pre,
                 W_ih, W_hh, b_lstm, W_dyn, b_dyn, W_lat, b_lat)


# speedup vs baseline: 1.1417x; 1.0246x over previous
import functools

import jax
import jax.numpy as jnp
from jax import lax
from jax.experimental import pallas as pl
from jax.experimental.pallas import tpu as pltpu
from jax.experimental.pallas import tpu_sc as plsc

N = 10000
DEG = 32
DYN = 8
LAT = 128
HID = 16
E = N * DEG

NC = 2
NS = 16
NW = NC * NS

CHUNK = 125
EPT = E // NW
NCHUNK = EPT // CHUNK
NBUF = 8
NG = NCHUNK // NBUF
NPAD = -(-N // (NS * 8)) * NS * 8
ZROWS = NPAD // NS
OTILES = 10
OROWS = N // OTILES

_mesh = plsc.VectorSubcoreMesh(core_axis_name="c", subcore_axis_name="s")


@functools.partial(
    pl.kernel,
    out_type=jax.ShapeDtypeStruct((NC, N, HID), jnp.float32),
    mesh=_mesh,
    scratch_types=[
        pltpu.VMEM((NCHUNK, CHUNK), jnp.int32),
        pltpu.VMEM((NCHUNK, CHUNK), jnp.int32),
        pltpu.VMEM((NBUF, CHUNK, HID), jnp.float32),
        pltpu.VMEM((ZROWS, HID), jnp.float32),
        pltpu.VMEM_SHARED((NPAD, HID), jnp.float32),
        [pltpu.SemaphoreType.DMA] * NBUF,
        [pltpu.SemaphoreType.DMA] * NBUF,
    ],
    compiler_params=pltpu.CompilerParams(use_tc_tiling_on_sc=False),
)
def _sc_route(p_hbm, gidx_hbm, didx_hbm, out_hbm,
              gidx_v, didx_v, rows_v, zbuf_v, acc_sh, gsems, ssems):
    c = lax.axis_index("c")
    s = lax.axis_index("s")
    w = s * NC + c
    pltpu.sync_copy(gidx_hbm.at[w], gidx_v)
    pltpu.sync_copy(didx_hbm.at[w], didx_v)
    def zrow(r, carry):
        zbuf_v[r, :] = jnp.zeros((HID,), jnp.float32)
        return carry
    lax.fori_loop(0, ZROWS, zrow, 0)
    pltpu.sync_copy(zbuf_v, acc_sh.at[pl.ds(s * ZROWS, ZROWS)])
    plsc.subcore_barrier()

    def gather_start(j, b):
        pltpu.async_copy(p_hbm.at[gidx_v.at[j]], rows_v.at[b], gsems[b])

    def gather_wait(j, b):
        pltpu.make_async_copy(p_hbm.at[gidx_v.at[j]], rows_v.at[b],
                              gsems[b]).wait()

    def scat_start(j, b):
        pltpu.async_copy(rows_v.at[b], acc_sh.at[didx_v.at[j]], ssems[b],
                         add=True)

    def scat_wait(j, b):
        pltpu.make_async_copy(rows_v.at[b], acc_sh.at[didx_v.at[j]],
                              ssems[b]).wait()

    for b in range(NBUF):
        gather_start(b, b)

    def group(i, carry):
        j0 = i * NBUF
        for b in range(NBUF):
            gather_wait(j0 + b, b)
            scat_start(j0 + b, b)
        for b in range(NBUF):
            scat_wait(j0 + b, b)
            gather_start(j0 + NBUF + b, b)
        return carry

    lax.fori_loop(0, NG - 1, group, 0)

    jl = (NG - 1) * NBUF
    for b in range(NBUF):
        gather_wait(jl + b, b)
        scat_start(jl + b, b)
    for b in range(NBUF):
        scat_wait(jl + b, b)

    plsc.subcore_barrier()
    @pl.when(s < OTILES)
    def _():
        pltpu.sync_copy(acc_sh.at[pl.ds(s * OROWS, OROWS)],
                        out_hbm.at[c, pl.ds(s * OROWS, OROWS)])


def _mm_body(x_ref, w_ref, o_ref):
    o_ref[...] = jnp.dot(x_ref[...], w_ref[...],
                         preferred_element_type=jnp.float32)


_BN1 = 1000


def _project(pk_lat_out, w_cat):
    return pl.pallas_call(
        _mm_body,
        grid=(N // _BN1,),
        in_specs=[
            pl.BlockSpec((_BN1, LAT), lambda i: (i, 0)),
            pl.BlockSpec((LAT, DEG * HID), lambda i: (0, 0)),
        ],
        out_specs=pl.BlockSpec((_BN1, DEG * HID), lambda i: (i, 0)),
        out_shape=jax.ShapeDtypeStruct((N, DEG * HID), jnp.float32),
    )(pk_lat_out, w_cat)


PK = 8
NR = N // PK


def _tail_body(acc_ref, dyn_ref, c_ref, h_ref, wd_ref, bpre_ref, wih_ref,
               whh_ref, blstm_ref, wdyn_ref, bdyn_ref, wlat_ref, blat_ref,
               dyn_out_ref, lat_out_ref, cnew_ref, hnew_ref):
    acc = acc_ref[0] + acc_ref[1]
    predyn = jnp.dot(dyn_ref[...], wd_ref[...],
                     preferred_element_type=jnp.float32)
    pre = jnp.tanh(acc + predyn + bpre_ref[...])
    gates = (jnp.dot(pre, wih_ref[...], preferred_element_type=jnp.float32)
             + jnp.dot(h_ref[...], whh_ref[...],
                       preferred_element_type=jnp.float32)
             + blstm_ref[...])
    i_g = jax.nn.sigmoid(gates[:, 0:LAT])
    f_g = jax.nn.sigmoid(gates[:, LAT:2 * LAT])
    g_g = jnp.tanh(gates[:, 2 * LAT:3 * LAT])
    o_g = jax.nn.sigmoid(gates[:, 3 * LAT:4 * LAT])
    c_new = f_g * c_ref[...] + i_g * g_g
    h_new = o_g * jnp.tanh(c_new)
    cnew_ref[...] = c_new
    hnew_ref[...] = h_new
    dyn_out_ref[...] = (jnp.dot(h_new, wdyn_ref[...],
                                preferred_element_type=jnp.float32)
                        + bdyn_ref[...])
    lat_out_ref[...] = jnp.tanh(jnp.dot(h_new, wlat_ref[...],
                                        preferred_element_type=jnp.float32)
                                + blat_ref[...])


def _tail(acc, dyn_in, pk_lstm_c, pk_lstm_h, w_pre_dyn, b_pre, w_ih, w_hh,
          b_lstm, w_dyn, b_dyn, w_lat, b_lat):
    eye = jnp.eye(PK, dtype=jnp.float32)
    wd_bd = jnp.einsum('rc,kj->krjc', w_pre_dyn, eye).reshape(PK * DYN,
                                                              PK * HID)
    w4 = w_ih.reshape(HID, 4, HID)
    wih_bd = jnp.einsum('rtc,kj->krtjc', w4, eye).reshape(PK * HID, 4 * LAT)
    w4h = w_hh.reshape(HID, 4, HID)
    whh_bd = jnp.einsum('rtc,kj->krtjc', w4h, eye).reshape(PK * HID, 4 * LAT)
    wdyn_bd = jnp.einsum('rc,kj->krjc', w_dyn, eye).reshape(PK * HID,
                                                            PK * DYN)
    wlat_bd = jnp.einsum('rc,kj->krjc', w_lat, eye).reshape(PK * HID,
                                                            PK * LAT)
    bpre_p = jnp.tile(b_pre, PK).reshape(1, PK * HID)
    blstm_p = jnp.tile(b_lstm.reshape(4, 1, HID),
                       (1, PK, 1)).reshape(1, 4 * LAT)
    bdyn_p = jnp.tile(b_dyn, PK).reshape(1, PK * DYN)
    blat_p = jnp.tile(b_lat, PK).reshape(1, PK * LAT)

    full = lambda r, c: pl.BlockSpec((r, c), lambda: (0, 0))
    outs = pl.pallas_call(
        _tail_body,
        in_specs=[
            pl.BlockSpec((NC, NR, PK * HID), lambda: (0, 0, 0)),
            full(NR, PK * DYN), full(NR, PK * HID), full(NR, PK * HID),
            full(PK * DYN, PK * HID), full(1, PK * HID),
            full(PK * HID, 4 * LAT), full(PK * HID, 4 * LAT),
            full(1, 4 * LAT), full(PK * HID, PK * DYN), full(1, PK * DYN),
            full(PK * HID, PK * LAT), full(1, PK * LAT),
        ],
        out_specs=[full(NR, PK * DYN), full(NR, PK * LAT),
                   full(NR, PK * HID), full(NR, PK * HID)],
        out_shape=[
            jax.ShapeDtypeStruct((NR, PK * DYN), jnp.float32),
            jax.ShapeDtypeStruct((NR, PK * LAT), jnp.float32),
            jax.ShapeDtypeStruct((NR, PK * HID), jnp.float32),
            jax.ShapeDtypeStruct((NR, PK * HID), jnp.float32),
        ],
    )(acc.reshape(NC, NR, PK * HID), dyn_in.reshape(NR, PK * DYN),
      pk_lstm_c.reshape(NR, PK * HID), pk_lstm_h.reshape(NR, PK * HID),
      wd_bd, bpre_p, wih_bd, whh_bd, blstm_p, wdyn_bd, bdyn_p,
      wlat_bd, blat_p)
    dyn_out, lat_out, c_new, h_new = outs
    return (dyn_out.reshape(N, DYN), lat_out.reshape(N, LAT),
            c_new.reshape(N, HID), h_new.reshape(N, HID))


def kernel(dyn_in, pk_lat_out, pk_lstm_c, pk_lstm_h, edge_src, edge_dst,
           edge_slot, W_pre, b_pre, W_ih, W_hh, b_lstm, W_dyn, b_dyn,
           W_lat, b_lat):
    w_cat = (W_pre[DYN:].reshape(DEG, LAT, HID)
             .transpose(1, 0, 2).reshape(LAT, DEG * HID))
    w_pre_dyn = W_pre[:DYN]

    gidx = (edge_src * DEG + edge_slot).reshape(NW, NCHUNK, CHUNK)
    didx = edge_dst.reshape(NW, NCHUNK, CHUNK)

    p = _project(pk_lat_out, w_cat)
    acc = _sc_route(p.reshape(N * DEG, HID), gidx, didx)
    return _tail(acc, dyn_in, pk_lstm_c, pk_lstm_h, w_pre_dyn, b_pre,
                 W_ih, W_hh, b_lstm, W_dyn, b_dyn, W_lat, b_lat)


# speedup vs baseline: 1.1675x; 1.0226x over previous
import functools

import jax
import jax.numpy as jnp
from jax import lax
from jax.experimental import pallas as pl
from jax.experimental.pallas import tpu as pltpu
from jax.experimental.pallas import tpu_sc as plsc

N = 10000
DEG = 32
DYN = 8
LAT = 128
HID = 16
E = N * DEG

NC = 2
NS = 16
NW = NC * NS

CHUNK = 125
EPT = E // NW
NCHUNK = EPT // CHUNK
NBUF = 10
NG = NCHUNK // NBUF
NPAD = -(-N // (NS * 8)) * NS * 8
ZROWS = NPAD // NS
OTILES = 10
OROWS = N // OTILES

_mesh = plsc.VectorSubcoreMesh(core_axis_name="c", subcore_axis_name="s")


@functools.partial(
    pl.kernel,
    out_type=jax.ShapeDtypeStruct((NC, N, HID), jnp.float32),
    mesh=_mesh,
    scratch_types=[
        pltpu.VMEM((NCHUNK, CHUNK), jnp.int32),
        pltpu.VMEM((NCHUNK, CHUNK), jnp.int32),
        pltpu.VMEM((NBUF, CHUNK, HID), jnp.float32),
        pltpu.VMEM((ZROWS, HID), jnp.float32),
        pltpu.VMEM_SHARED((NPAD, HID), jnp.float32),
        [pltpu.SemaphoreType.DMA] * NBUF,
        [pltpu.SemaphoreType.DMA] * NBUF,
    ],
    compiler_params=pltpu.CompilerParams(use_tc_tiling_on_sc=False),
)
def _sc_route(p_hbm, gidx_hbm, didx_hbm, out_hbm,
              gidx_v, didx_v, rows_v, zbuf_v, acc_sh, gsems, ssems):
    c = lax.axis_index("c")
    s = lax.axis_index("s")
    w = s * NC + c
    pltpu.sync_copy(gidx_hbm.at[w], gidx_v)
    pltpu.sync_copy(didx_hbm.at[w], didx_v)
    def zrow(r, carry):
        zbuf_v[r, :] = jnp.zeros((HID,), jnp.float32)
        return carry
    lax.fori_loop(0, ZROWS, zrow, 0)
    pltpu.sync_copy(zbuf_v, acc_sh.at[pl.ds(s * ZROWS, ZROWS)])
    plsc.subcore_barrier()

    def gather_start(j, b):
        pltpu.async_copy(p_hbm.at[gidx_v.at[j]], rows_v.at[b], gsems[b])

    def gather_wait(j, b):
        pltpu.make_async_copy(p_hbm.at[gidx_v.at[j]], rows_v.at[b],
                              gsems[b]).wait()

    def scat_start(j, b):
        pltpu.async_copy(rows_v.at[b], acc_sh.at[didx_v.at[j]], ssems[b],
                         add=True)

    def scat_wait(j, b):
        pltpu.make_async_copy(rows_v.at[b], acc_sh.at[didx_v.at[j]],
                              ssems[b]).wait()

    for b in range(NBUF):
        gather_start(b, b)

    def group(i, carry):
        j0 = i * NBUF
        for b in range(NBUF):
            gather_wait(j0 + b, b)
            scat_start(j0 + b, b)
        for b in range(NBUF):
            scat_wait(j0 + b, b)
            gather_start(j0 + NBUF + b, b)
        return carry

    lax.fori_loop(0, NG - 1, group, 0)

    jl = (NG - 1) * NBUF
    for b in range(NBUF):
        gather_wait(jl + b, b)
        scat_start(jl + b, b)
    for b in range(NBUF):
        scat_wait(jl + b, b)

    plsc.subcore_barrier()
    @pl.when(s < OTILES)
    def _():
        pltpu.sync_copy(acc_sh.at[pl.ds(s * OROWS, OROWS)],
                        out_hbm.at[c, pl.ds(s * OROWS, OROWS)])


def _mm_body(x_ref, w_ref, o_ref):
    o_ref[...] = jnp.dot(x_ref[...], w_ref[...],
                         preferred_element_type=jnp.float32)


_BN1 = 2000


def _project(pk_lat_out, w_cat):
    return pl.pallas_call(
        _mm_body,
        grid=(N // _BN1,),
        in_specs=[
            pl.BlockSpec((_BN1, LAT), lambda i: (i, 0)),
            pl.BlockSpec((LAT, DEG * HID), lambda i: (0, 0)),
        ],
        out_specs=pl.BlockSpec((_BN1, DEG * HID), lambda i: (i, 0)),
        out_shape=jax.ShapeDtypeStruct((N, DEG * HID), jnp.float32),
    )(pk_lat_out, w_cat)


PK = 8
NR = N // PK


def _tail_body(acc_ref, dyn_ref, c_ref, h_ref, wd_ref, bpre_ref, wih_ref,
               whh_ref, blstm_ref, wdyn_ref, bdyn_ref, wlat_ref, blat_ref,
               dyn_out_ref, lat_out_ref, cnew_ref, hnew_ref):
    acc = acc_ref[0] + acc_ref[1]
    predyn = jnp.dot(dyn_ref[...], wd_ref[...],
                     preferred_element_type=jnp.float32)
    pre = jnp.tanh(acc + predyn + bpre_ref[...])
    gates = (jnp.dot(pre, wih_ref[...], preferred_element_type=jnp.float32)
             + jnp.dot(h_ref[...], whh_ref[...],
                       preferred_element_type=jnp.float32)
             + blstm_ref[...])
    i_g = jax.nn.sigmoid(gates[:, 0:LAT])
    f_g = jax.nn.sigmoid(gates[:, LAT:2 * LAT])
    g_g = jnp.tanh(gates[:, 2 * LAT:3 * LAT])
    o_g = jax.nn.sigmoid(gates[:, 3 * LAT:4 * LAT])
    c_new = f_g * c_ref[...] + i_g * g_g
    h_new = o_g * jnp.tanh(c_new)
    cnew_ref[...] = c_new
    hnew_ref[...] = h_new
    dyn_out_ref[...] = (jnp.dot(h_new, wdyn_ref[...],
                                preferred_element_type=jnp.float32)
                        + bdyn_ref[...])
    lat_out_ref[...] = jnp.tanh(jnp.dot(h_new, wlat_ref[...],
                                        preferred_element_type=jnp.float32)
                                + blat_ref[...])


def _tail(acc, dyn_in, pk_lstm_c, pk_lstm_h, w_pre_dyn, b_pre, w_ih, w_hh,
          b_lstm, w_dyn, b_dyn, w_lat, b_lat):
    eye = jnp.eye(PK, dtype=jnp.float32)
    wd_bd = jnp.einsum('rc,kj->krjc', w_pre_dyn, eye).reshape(PK * DYN,
                                                              PK * HID)
    w4 = w_ih.reshape(HID, 4, HID)
    wih_bd = jnp.einsum('rtc,kj->krtjc', w4, eye).reshape(PK * HID, 4 * LAT)
    w4h = w_hh.reshape(HID, 4, HID)
    whh_bd = jnp.einsum('rtc,kj->krtjc', w4h, eye).reshape(PK * HID, 4 * LAT)
    wdyn_bd = jnp.einsum('rc,kj->krjc', w_dyn, eye).reshape(PK * HID,
                                                            PK * DYN)
    wlat_bd = jnp.einsum('rc,kj->krjc', w_lat, eye).reshape(PK * HID,
                                                            PK * LAT)
    bpre_p = jnp.tile(b_pre, PK).reshape(1, PK * HID)
    blstm_p = jnp.tile(b_lstm.reshape(4, 1, HID),
                       (1, PK, 1)).reshape(1, 4 * LAT)
    bdyn_p = jnp.tile(b_dyn, PK).reshape(1, PK * DYN)
    blat_p = jnp.tile(b_lat, PK).reshape(1, PK * LAT)

    full = lambda r, c: pl.BlockSpec((r, c), lambda: (0, 0))
    outs = pl.pallas_call(
        _tail_body,
        in_specs=[
            pl.BlockSpec((NC, NR, PK * HID), lambda: (0, 0, 0)),
            full(NR, PK * DYN), full(NR, PK * HID), full(NR, PK * HID),
            full(PK * DYN, PK * HID), full(1, PK * HID),
            full(PK * HID, 4 * LAT), full(PK * HID, 4 * LAT),
            full(1, 4 * LAT), full(PK * HID, PK * DYN), full(1, PK * DYN),
            full(PK * HID, PK * LAT), full(1, PK * LAT),
        ],
        out_specs=[full(NR, PK * DYN), full(NR, PK * LAT),
                   full(NR, PK * HID), full(NR, PK * HID)],
        out_shape=[
            jax.ShapeDtypeStruct((NR, PK * DYN), jnp.float32),
            jax.ShapeDtypeStruct((NR, PK * LAT), jnp.float32),
            jax.ShapeDtypeStruct((NR, PK * HID), jnp.float32),
            jax.ShapeDtypeStruct((NR, PK * HID), jnp.float32),
        ],
    )(acc.reshape(NC, NR, PK * HID), dyn_in.reshape(NR, PK * DYN),
      pk_lstm_c.reshape(NR, PK * HID), pk_lstm_h.reshape(NR, PK * HID),
      wd_bd, bpre_p, wih_bd, whh_bd, blstm_p, wdyn_bd, bdyn_p,
      wlat_bd, blat_p)
    dyn_out, lat_out, c_new, h_new = outs
    return (dyn_out.reshape(N, DYN), lat_out.reshape(N, LAT),
            c_new.reshape(N, HID), h_new.reshape(N, HID))


def kernel(dyn_in, pk_lat_out, pk_lstm_c, pk_lstm_h, edge_src, edge_dst,
           edge_slot, W_pre, b_pre, W_ih, W_hh, b_lstm, W_dyn, b_dyn,
           W_lat, b_lat):
    w_cat = (W_pre[DYN:].reshape(DEG, LAT, HID)
             .transpose(1, 0, 2).reshape(LAT, DEG * HID))
    w_pre_dyn = W_pre[:DYN]

    gidx = (edge_src * DEG + edge_slot).reshape(NW, NCHUNK, CHUNK)
    didx = edge_dst.reshape(NW, NCHUNK, CHUNK)

    p = _project(pk_lat_out, w_cat)
    acc = _sc_route(p.reshape(N * DEG, HID), gidx, didx)
    return _tail(acc, dyn_in, pk_lstm_c, pk_lstm_h, w_pre_dyn, b_pre,
                 W_ih, W_hh, b_lstm, W_dyn, b_dyn, W_lat, b_lat)
